# baseline (speedup 1.0000x reference)
import functools

import jax
import jax.numpy as jnp
from jax import lax
from jax.experimental import pallas as pl
from jax.experimental.pallas import tpu as pltpu
from jax.experimental.pallas import tpu_sc as plsc

H, DH, E, K = 12, 64, 8, 2
DHE = DH + 8
SCALE = DH ** -0.5
SLAB = 128
NSUB = 16
SC_L = 16


def _gate_kernel(x_ref, w2_ref, g_ref):
    xt = x_ref[:, :].T
    y2 = lax.dot_general(w2_ref[:, :], xt, (((0,), (0,)), ((), ())),
                         preferred_element_type=jnp.float32)
    for w in range(g_ref.shape[0]):
        g_ref[w, :, :] = y2[:, w * SLAB:(w + 1) * SLAB]


def _route_slab(g_v, o_v, weighted):
    one = jnp.full((SC_L,), 1.0, jnp.float32)
    ninf = jnp.full((SC_L,), -jnp.inf, jnp.float32)

    @pl.loop(0, SLAB // SC_L)
    def _chunks(c):
        sl = pl.ds(c * SC_L, SC_L)
        for h in range(H):
            g = [g_v[h * E + e, sl] for e in range(E)]
            m1 = g[0]
            for e in range(1, E):
                m1 = jnp.maximum(m1, g[e])
            is1, free = [], one
            for e in range(E):
                eq = jnp.where(g[e] == m1, one, 0.0)
                hit = eq * free
                is1.append(hit)
                free = free * (one - eq)
            g2 = [jnp.where(is1[e] > 0.0, ninf, g[e]) for e in range(E)]
            m2 = g2[0]
            for e in range(1, E):
                m2 = jnp.maximum(m2, g2[e])
            is2, free2 = [], one
            for e in range(E):
                eq = jnp.where(g2[e] == m2, one, 0.0)
                is2.append(eq * free2)
                free2 = free2 * (one - eq)
            for e in range(E):
                sel = is1[e] + is2[e]
                if weighted:
                    o_v[h * E + e, sl] = sel / (1.0 + jnp.exp(-g[e]))
                else:
                    o_v[h * E + e, sl] = sel


def _route_sc_kernel(g_hbm, coef_hbm, cnt_hbm, g_v, o_v):
    cid = lax.axis_index("c")
    sid = lax.axis_index("s")

    @pl.when(cid == 0)
    def _coef():
        pltpu.sync_copy(g_hbm.at[sid, pl.ds(0, H * E)], g_v)
        _route_slab(g_v, o_v, True)
        pltpu.sync_copy(o_v, coef_hbm.at[sid])

    @pl.when(cid == 1)
    def _cnt():
        pltpu.sync_copy(g_hbm.at[sid, pl.ds(H * E, H * E)], g_v)
        _route_slab(g_v, o_v, False)
        pltpu.sync_copy(o_v, cnt_hbm.at[sid])


def _proj_kernel(x_ref, w1_ref, q_ref, k_ref, xv_ref):
    xt = x_ref[:, :].T
    d = x_ref.shape[1]
    y1 = lax.dot_general(w1_ref[:, :], xt.astype(jnp.bfloat16),
                         (((0,), (0,)), ((), ())),
                         preferred_element_type=jnp.float32)
    for h in range(H):
        q_ref[h, :, :] = (y1[h * DH:(h + 1) * DH, :] * SCALE).astype(jnp.bfloat16)
        k_ref[h, :, :] = y1[d + h * DH:d + (h + 1) * DH, :].astype(jnp.bfloat16)
    xv_ref[:, :] = y1[2 * d:2 * d + E * DH, :].astype(jnp.bfloat16)


def _vcomb_kernel(xv_ref, coef_ref, v_ref):
    tb = xv_ref.shape[1]
    cf = jnp.concatenate([coef_ref[w] for w in range(coef_ref.shape[0])],
                         axis=1)
    xv = xv_ref[:, :].astype(jnp.float32)
    for h in range(H):
        vh = cf[h * E:h * E + 1, :] * xv[0:DH, :]
        for e in range(1, E):
            vh = vh + cf[h * E + e:h * E + e + 1, :] * xv[e * DH:(e + 1) * DH, :]
        v_ref[h, :DH, :] = vh.astype(jnp.bfloat16)
        v_ref[h, DH:DH + 1, :] = jnp.ones((1, tb), jnp.bfloat16)
        v_ref[h, DH + 1:, :] = jnp.zeros((DHE - DH - 1, tb), jnp.bfloat16)


def _attn_out_kernel(q_ref, k_ref, v_ref, cnt_ref, wo_ref, res_ref,
                     z_ref, cnt_s):
    h = pl.program_id(1)

    @pl.when(h == 0)
    def _stage_cnt():
        cnt_s[:, :] = jnp.concatenate(
            [cnt_ref[w] for w in range(cnt_ref.shape[0])], axis=1)

    s = lax.dot_general(q_ref[0], k_ref[0], (((0,), (0,)), ((), ())),
                        preferred_element_type=jnp.float32)
    m = jnp.max(s, axis=1, keepdims=True)
    p = jnp.exp((s - m).astype(jnp.bfloat16))
    ov = lax.dot_general(v_ref[0], p, (((1,), (1,)), ((), ())),
                         preferred_element_type=jnp.float32)
    ot = ov[:DH, :] * (1.0 / ov[DH:DH + 1, :])
    c_h = cnt_s[pl.ds(h * E, E), :]

    @pl.when(h == 0)
    def _init():
        for e in range(E):
            z_ref[e * DH:(e + 1) * DH, :] = c_h[e:e + 1, :] * ot

    @pl.when(h != 0)
    def _acc():
        for e in range(E):
            z_ref[e * DH:(e + 1) * DH, :] += c_h[e:e + 1, :] * ot

    @pl.when(h == H - 1)
    def _fin():
        res_ref[:, :] = lax.dot_general(
            z_ref[:, :].astype(jnp.bfloat16), wo_ref[:, :],
            (((0,), (0,)), ((), ())), preferred_element_type=jnp.float32)


def kernel(x, Wq, Wk, Ws, Wd, Wv, Wo):
    b, t, d = x.shape
    x2 = x.reshape(t, d)
    nslab = t // SLAB
    wv_flat = Wv.transpose(1, 0, 2).reshape(d, E * DH)
    w1 = jnp.concatenate([Wq, Wk, wv_flat], axis=1).astype(jnp.bfloat16)
    w2 = jnp.concatenate([Ws, Wd], axis=1)
    wo_flat = Wo.reshape(E * DH, d).astype(jnp.bfloat16)

    tb_a = 512
    spw = tb_a // SLAB
    gates = pl.pallas_call(
        _gate_kernel,
        grid=(t // tb_a,),
        in_specs=[
            pl.BlockSpec((tb_a, d), lambda i: (i, 0)),
            pl.BlockSpec((d, 2 * H * E), lambda i: (0, 0)),
        ],
        out_specs=pl.BlockSpec((spw, 2 * H * E, SLAB), lambda i: (i, 0, 0)),
        out_shape=jax.ShapeDtypeStruct((nslab, 2 * H * E, SLAB), jnp.float32),
    )(x2, w2)

    route = functools.partial(
        pl.kernel,
        mesh=plsc.VectorSubcoreMesh(core_axis_name="c", subcore_axis_name="s"),
        out_type=[
            jax.ShapeDtypeStruct((nslab, H * E, SLAB), jnp.float32),
            jax.ShapeDtypeStruct((nslab, H * E, SLAB), jnp.float32),
        ],
        scratch_types=[
            pltpu.VMEM((H * E, SLAB), jnp.float32),
            pltpu.VMEM((H * E, SLAB), jnp.float32),
        ],
    )(_route_sc_kernel)
    coef, cnt = route(gates)

    q, k, xv = pl.pallas_call(
        _proj_kernel,
        grid=(t // tb_a,),
        in_specs=[
            pl.BlockSpec((tb_a, d), lambda i: (i, 0)),
            pl.BlockSpec((d, 2 * d + E * DH), lambda i: (0, 0)),
        ],
        out_specs=[
            pl.BlockSpec((H, DH, tb_a), lambda i: (0, 0, i)),
            pl.BlockSpec((H, DH, tb_a), lambda i: (0, 0, i)),
            pl.BlockSpec((E * DH, tb_a), lambda i: (0, i)),
        ],
        out_shape=[
            jax.ShapeDtypeStruct((H, DH, t), jnp.bfloat16),
            jax.ShapeDtypeStruct((H, DH, t), jnp.bfloat16),
            jax.ShapeDtypeStruct((E * DH, t), jnp.bfloat16),
        ],
    )(x2, w1)

    v = pl.pallas_call(
        _vcomb_kernel,
        grid=(t // tb_a,),
        in_specs=[
            pl.BlockSpec((E * DH, tb_a), lambda i: (0, i)),
            pl.BlockSpec((spw, H * E, SLAB), lambda i: (i, 0, 0)),
        ],
        out_specs=pl.BlockSpec((H, DHE, tb_a), lambda i: (0, 0, i)),
        out_shape=jax.ShapeDtypeStruct((H, DHE, t), jnp.bfloat16),
    )(xv, coef)

    tb_q = 1024
    qpw = tb_q // SLAB
    res = pl.pallas_call(
        _attn_out_kernel,
        grid=(t // tb_q, H),
        in_specs=[
            pl.BlockSpec((1, DH, tb_q), lambda i, h: (h, 0, i)),
            pl.BlockSpec((1, DH, t), lambda i, h: (h, 0, 0)),
            pl.BlockSpec((1, DHE, t), lambda i, h: (h, 0, 0)),
            pl.BlockSpec((qpw, H * E, SLAB), lambda i, h: (i, 0, 0)),
            pl.BlockSpec((E * DH, d), lambda i, h: (0, 0)),
        ],
        out_specs=pl.BlockSpec((tb_q, d), lambda i, h: (i, 0)),
        out_shape=jax.ShapeDtypeStruct((t, d), jnp.float32),
        scratch_shapes=[pltpu.VMEM((E * DH, tb_q), jnp.float32),
                        pltpu.VMEM((H * E, tb_q), jnp.float32)],
    )(q, k, v, cnt, wo_flat)

    return res.reshape(b, t, d)


# speedup vs baseline: 1.2364x; 1.2364x over previous
import jax
import jax.numpy as jnp
from jax import lax
from jax.experimental import pallas as pl
from jax.experimental.pallas import tpu as pltpu

H, DH, E, K = 12, 64, 8, 2
DHE = DH + 8
SCALE = DH ** -0.5


def _top2_sel_t(g):
    tb = g.shape[1]
    iota = lax.broadcasted_iota(jnp.int32, (E, tb), 0)
    m1 = jnp.max(g, axis=0, keepdims=True)
    i1 = jnp.min(jnp.where(g == m1, iota, E), axis=0, keepdims=True)
    sel1 = iota == i1
    g2 = jnp.where(sel1, -jnp.inf, g)
    m2 = jnp.max(g2, axis=0, keepdims=True)
    i2 = jnp.min(jnp.where(g2 == m2, iota, E), axis=0, keepdims=True)
    sel2 = iota == i2
    return sel1, sel2


def _proj_kernel(x_ref, w1_ref, w2_ref, q_ref, k_ref, v_ref, cnt_ref):
    xt = x_ref[:, :].T
    d = x_ref.shape[1]
    tb = xt.shape[1]
    y1 = lax.dot_general(w1_ref[:, :], xt.astype(jnp.bfloat16),
                         (((0,), (0,)), ((), ())),
                         preferred_element_type=jnp.float32)
    y2 = lax.dot_general(w2_ref[:, :], xt, (((0,), (0,)), ((), ())),
                         preferred_element_type=jnp.float32)
    xv = y1[2 * d:2 * d + E * DH, :].astype(jnp.bfloat16)
    q_ref[:, :, :] = y1[:d, :].astype(jnp.bfloat16).reshape(H, DH, tb)
    k_ref[:, :, :] = y1[d:2 * d, :].astype(jnp.bfloat16).reshape(H, DH, tb)
    for h in range(H):
        gv = y2[h * E:(h + 1) * E, :]
        sel1, sel2 = _top2_sel_t(gv)
        coef = (jax.nn.sigmoid(gv) *
                (sel1 | sel2).astype(jnp.float32)).astype(jnp.bfloat16)
        vh = coef[0:1, :] * xv[0:DH, :]
        for e in range(1, E):
            vh = vh + coef[e:e + 1, :] * xv[e * DH:(e + 1) * DH, :]
        v_ref[h, :DH, :] = vh
        v_ref[h, DH:DH + 1, :] = jnp.ones((1, tb), jnp.bfloat16)
        v_ref[h, DH + 1:, :] = jnp.zeros((DHE - DH - 1, tb), jnp.bfloat16)
        go = y2[H * E + h * E:H * E + (h + 1) * E, :]
        o1, o2 = _top2_sel_t(go)
        cnt_ref[h * E:(h + 1) * E, :] = (o1 | o2).astype(jnp.float32)


def _attn_out_kernel(q_ref, k_ref, v_ref, cnt_ref, wo_ref, res_ref, z_ref):
    h = pl.program_id(1)
    s = lax.dot_general(q_ref[0], k_ref[0], (((0,), (0,)), ((), ())),
                        preferred_element_type=jnp.float32)
    m = jnp.max(s, axis=1, keepdims=True)
    p = jnp.exp((s - m).astype(jnp.bfloat16))
    ov = lax.dot_general(v_ref[0], p, (((1,), (1,)), ((), ())),
                         preferred_element_type=jnp.float32)
    ot = ov[:DH, :] * (1.0 / ov[DH:DH + 1, :])
    c_h = cnt_ref[pl.ds(h * E, E), :]

    @pl.when(h == 0)
    def _init():
        for e in range(E):
            z_ref[e * DH:(e + 1) * DH, :] = c_h[e:e + 1, :] * ot

    @pl.when(h != 0)
    def _acc():
        for e in range(E):
            z_ref[e * DH:(e + 1) * DH, :] += c_h[e:e + 1, :] * ot

    @pl.when(h == H - 1)
    def _fin():
        res_ref[:, :] = lax.dot_general(
            z_ref[:, :].astype(jnp.bfloat16), wo_ref[:, :],
            (((0,), (0,)), ((), ())), preferred_element_type=jnp.float32)


def kernel(x, Wq, Wk, Ws, Wd, Wv, Wo):
    b, t, d = x.shape
    x2 = x.reshape(t, d)
    wv_flat = Wv.transpose(1, 0, 2).reshape(d, E * DH)
    w1 = jnp.concatenate([Wq * SCALE, Wk, wv_flat], axis=1).astype(jnp.bfloat16)
    w2 = jnp.concatenate([Ws, Wd], axis=1)
    wo_flat = Wo.reshape(E * DH, d).astype(jnp.bfloat16)

    tb_a = 512
    q, k, v, cnt = pl.pallas_call(
        _proj_kernel,
        grid=(t // tb_a,),
        in_specs=[
            pl.BlockSpec((tb_a, d), lambda i: (i, 0)),
            pl.BlockSpec((d, 2 * d + E * DH), lambda i: (0, 0)),
            pl.BlockSpec((d, 2 * H * E), lambda i: (0, 0)),
        ],
        out_specs=[
            pl.BlockSpec((H, DH, tb_a), lambda i: (0, 0, i)),
            pl.BlockSpec((H, DH, tb_a), lambda i: (0, 0, i)),
            pl.BlockSpec((H, DHE, tb_a), lambda i: (0, 0, i)),
            pl.BlockSpec((H * E, tb_a), lambda i: (0, i)),
        ],
        out_shape=[
            jax.ShapeDtypeStruct((H, DH, t), jnp.bfloat16),
            jax.ShapeDtypeStruct((H, DH, t), jnp.bfloat16),
            jax.ShapeDtypeStruct((H, DHE, t), jnp.bfloat16),
            jax.ShapeDtypeStruct((H * E, t), jnp.float32),
        ],
    )(x2, w1, w2)

    tb_q = 1024
    res = pl.pallas_call(
        _attn_out_kernel,
        grid=(t // tb_q, H),
        in_specs=[
            pl.BlockSpec((1, DH, tb_q), lambda i, h: (h, 0, i)),
            pl.BlockSpec((1, DH, t), lambda i, h: (h, 0, 0)),
            pl.BlockSpec((1, DHE, t), lambda i, h: (h, 0, 0)),
            pl.BlockSpec((H * E, tb_q), lambda i, h: (0, i)),
            pl.BlockSpec((E * DH, d), lambda i, h: (0, 0)),
        ],
        out_specs=pl.BlockSpec((tb_q, d), lambda i, h: (i, 0)),
        out_shape=jax.ShapeDtypeStruct((t, d), jnp.float32),
        scratch_shapes=[pltpu.VMEM((E * DH, tb_q), jnp.float32)],
    )(q, k, v, cnt, wo_flat)

    return res.reshape(b, t, d)


# speedup vs baseline: 1.2928x; 1.0456x over previous
import jax
import jax.numpy as jnp
from jax import lax
from jax.experimental import pallas as pl
from jax.experimental.pallas import tpu as pltpu

H, DH, E, K = 12, 64, 8, 2
DHE = DH + 8
SCALE = DH ** -0.5


def _top2_sel_t(g):
    tb = g.shape[1]
    iota = lax.broadcasted_iota(jnp.int32, (E, tb), 0)
    m1 = jnp.max(g, axis=0, keepdims=True)
    i1 = jnp.min(jnp.where(g == m1, iota, E), axis=0, keepdims=True)
    sel1 = iota == i1
    g2 = jnp.where(sel1, -jnp.inf, g)
    m2 = jnp.max(g2, axis=0, keepdims=True)
    i2 = jnp.min(jnp.where(g2 == m2, iota, E), axis=0, keepdims=True)
    sel2 = iota == i2
    return sel1, sel2


def _proj_kernel(x_ref, w1_ref, w2_ref, q_ref, k_ref, v_ref, cnt_ref):
    xt = x_ref[:, :].T
    d = x_ref.shape[1]
    tb = xt.shape[1]
    y1 = lax.dot_general(w1_ref[:, :], xt.astype(jnp.bfloat16),
                         (((0,), (0,)), ((), ())),
                         preferred_element_type=jnp.float32)
    y2 = lax.dot_general(w2_ref[:, :], xt, (((0,), (0,)), ((), ())),
                         preferred_element_type=jnp.float32)
    xv = y1[2 * d:2 * d + E * DH, :].astype(jnp.bfloat16)
    q_ref[:, :, :] = y1[:d, :].astype(jnp.bfloat16).reshape(H, DH, tb)
    k_ref[:, :, :] = y1[d:2 * d, :].astype(jnp.bfloat16).reshape(H, DH, tb)
    for h in range(H):
        gv = y2[h * E:(h + 1) * E, :]
        sel1, sel2 = _top2_sel_t(gv)
        coef = (jax.nn.sigmoid(gv) *
                (sel1 | sel2).astype(jnp.float32)).astype(jnp.bfloat16)
        vh = coef[0:1, :] * xv[0:DH, :]
        for e in range(1, E):
            vh = vh + coef[e:e + 1, :] * xv[e * DH:(e + 1) * DH, :]
        v_ref[h, :DH, :] = vh
        v_ref[h, DH:DH + 1, :] = jnp.ones((1, tb), jnp.bfloat16)
        v_ref[h, DH + 1:, :] = jnp.zeros((DHE - DH - 1, tb), jnp.bfloat16)
        go = y2[H * E + h * E:H * E + (h + 1) * E, :]
        o1, o2 = _top2_sel_t(go)
        cnt_ref[h * E:(h + 1) * E, :] = (o1 | o2).astype(jnp.float32)


def _attn_out_kernel(q_ref, k_ref, v_ref, cnt_ref, wo_ref, res_ref, z_ref):
    h = pl.program_id(1)
    s = lax.dot_general(q_ref[0], k_ref[0], (((0,), (0,)), ((), ())),
                        preferred_element_type=jnp.float32)
    m = jnp.max(s, axis=1, keepdims=True)
    p = jnp.exp((s - m).astype(jnp.bfloat16))
    ov = lax.dot_general(v_ref[0], p, (((1,), (1,)), ((), ())),
                         preferred_element_type=jnp.float32)
    ot = ov[:DH, :] * (1.0 / ov[DH:DH + 1, :])
    c_h = cnt_ref[pl.ds(h * E, E), :]

    @pl.when(h == 0)
    def _init():
        for e in range(E):
            z_ref[e * DH:(e + 1) * DH, :] = c_h[e:e + 1, :] * ot

    @pl.when(h != 0)
    def _acc():
        for e in range(E):
            z_ref[e * DH:(e + 1) * DH, :] += c_h[e:e + 1, :] * ot

    @pl.when(h == H - 1)
    def _fin():
        res_ref[:, :] = lax.dot_general(
            z_ref[:, :].astype(jnp.bfloat16), wo_ref[:, :],
            (((0,), (0,)), ((), ())), preferred_element_type=jnp.float32)


def kernel(x, Wq, Wk, Ws, Wd, Wv, Wo):
    b, t, d = x.shape
    x2 = x.reshape(t, d)
    wv_flat = Wv.transpose(1, 0, 2).reshape(d, E * DH)
    w1 = jnp.concatenate([Wq * SCALE, Wk, wv_flat], axis=1).astype(jnp.bfloat16)
    w2 = jnp.concatenate([Ws, Wd], axis=1)
    wo_flat = Wo.reshape(E * DH, d).astype(jnp.bfloat16)

    tb_a = 1024
    q, k, v, cnt = pl.pallas_call(
        _proj_kernel,
        grid=(t // tb_a,),
        in_specs=[
            pl.BlockSpec((tb_a, d), lambda i: (i, 0)),
            pl.BlockSpec((d, 2 * d + E * DH), lambda i: (0, 0)),
            pl.BlockSpec((d, 2 * H * E), lambda i: (0, 0)),
        ],
        out_specs=[
            pl.BlockSpec((H, DH, tb_a), lambda i: (0, 0, i)),
            pl.BlockSpec((H, DH, tb_a), lambda i: (0, 0, i)),
            pl.BlockSpec((H, DHE, tb_a), lambda i: (0, 0, i)),
            pl.BlockSpec((H * E, tb_a), lambda i: (0, i)),
        ],
        out_shape=[
            jax.ShapeDtypeStruct((H, DH, t), jnp.bfloat16),
            jax.ShapeDtypeStruct((H, DH, t), jnp.bfloat16),
            jax.ShapeDtypeStruct((H, DHE, t), jnp.bfloat16),
            jax.ShapeDtypeStruct((H * E, t), jnp.float32),
        ],
    )(x2, w1, w2)

    tb_q = 2048
    res = pl.pallas_call(
        _attn_out_kernel,
        grid=(t // tb_q, H),
        in_specs=[
            pl.BlockSpec((1, DH, tb_q), lambda i, h: (h, 0, i)),
            pl.BlockSpec((1, DH, t), lambda i, h: (h, 0, 0)),
            pl.BlockSpec((1, DHE, t), lambda i, h: (h, 0, 0)),
            pl.BlockSpec((H * E, tb_q), lambda i, h: (0, i)),
            pl.BlockSpec((E * DH, d), lambda i, h: (0, 0)),
        ],
        out_specs=pl.BlockSpec((tb_q, d), lambda i, h: (i, 0)),
        out_shape=jax.ShapeDtypeStruct((t, d), jnp.float32),
        scratch_shapes=[pltpu.VMEM((E * DH, tb_q), jnp.float32)],
    )(q, k, v, cnt, wo_flat)

    return res.reshape(b, t, d)
